# baseline (speedup 1.0000x reference)
import functools

import jax
import jax.numpy as jnp
from jax import lax
from jax.experimental import pallas as pl
from jax.experimental.pallas import tpu as pltpu
from jax.experimental.pallas import tpu_sc as plsc

B, D, H, O, E = 2048, 1024, 2048, 1024, 8
K = 2
BT = 256
NB = (B * K) // BT + E
CT = NB * BT
NC, NS, L = 2, 16, 16
NW = NC * NS
SL = CT // NW
TB = B // NW
CCH = TB // 2
GB = 256
DW = D // 2


def _gate_body(*refs):
    x_refs = refs[:E]
    gw_ref, gb_ref, lg_ref = refs[E], refs[E + 1], refs[E + 2]
    acc = gb_ref[...][None, :]
    for j in range(E):
        gw = gw_ref[:, j * D:(j + 1) * D]
        acc = acc + lax.dot_general(x_refs[j][...], gw,
                                    (((1,), (1,)), ((), ())),
                                    preferred_element_type=jnp.float32)
    lg_ref[...] = acc


def _route_body(lg_ref, pos_ref, srcv_ref, wv_ref, eob_ref, used_ref):
    l = lg_ref[...]
    cols = lax.broadcasted_iota(jnp.int32, (B, E), 1)
    a1 = jnp.argmax(l, axis=1)[:, None]
    m1 = jnp.max(l, axis=1)[:, None]
    lmask = jnp.where(cols == a1, -jnp.inf, l)
    a2 = jnp.argmax(lmask, axis=1)[:, None]
    m2 = jnp.max(lmask, axis=1)[:, None]
    z = jnp.exp(m2 - m1)
    w1 = 1.0 / (1.0 + z)
    w2 = 1.0 - w1
    msk1 = (cols == a1).astype(jnp.float32)
    msk2 = (cols == a2).astype(jnp.float32)
    m = msk1 + msk2
    rows_i = lax.broadcasted_iota(jnp.int32, (B, B), 0)
    cols_i = lax.broadcasted_iota(jnp.int32, (B, B), 1)
    lt = (rows_i > cols_i).astype(jnp.float32)
    rank = lax.dot_general(lt, m, (((1,), (0,)), ((), ())),
                           preferred_element_type=jnp.float32)
    counts = jnp.sum(m, axis=0)[None, :]
    nb = jnp.floor((counts + (BT - 1)) * (1.0 / BT))
    r8 = lax.broadcasted_iota(jnp.int32, (E, E), 0)
    c8 = lax.broadcasted_iota(jnp.int32, (E, E), 1)
    t8 = (r8 <= c8).astype(jnp.float32)
    cum = lax.dot_general(nb, t8, (((1,), (0,)), ((), ())),
                          preferred_element_type=jnp.float32)
    off = (cum - nb) * float(BT)
    pos_e = off + rank
    pos0 = jnp.sum(msk1 * pos_e, axis=1)[:, None]
    pos1 = jnp.sum(msk2 * pos_e, axis=1)[:, None]
    pos_ref[...] = jnp.concatenate(
        [pos0.astype(jnp.int32), pos1.astype(jnp.int32)], axis=1)
    t_iota = lax.broadcasted_iota(jnp.int32, (B, 1), 0)
    srcv_ref[...] = jnp.concatenate(
        [a1 * B + t_iota, a2 * B + t_iota], axis=1)
    wv_ref[...] = jnp.concatenate([w1, w2], axis=1)
    g_iota = lax.broadcasted_iota(jnp.int32, (1, NB), 1).astype(jnp.float32)
    eob = jnp.zeros((1, NB), jnp.float32)
    for j in range(E - 1):
        cj = lax.slice(cum, (0, j), (1, j + 1))
        eob = eob + (g_iota >= cj).astype(jnp.float32)
    eob_ref[...] = eob.astype(jnp.int32)
    used_ref[...] = lax.slice(cum, (0, E - 1), (1, E)).astype(jnp.int32)


def _route_gather_body(pos_hbm, srcv_hbm, w_hbm, xf_hbm, xg_hbm, sw_hbm,
                       pos_v, srcv_v, w_v, idx_v, sw_v, rows_v, sem):
    wid = lax.axis_index("s") * NC + lax.axis_index("c")
    base = wid * SL
    pltpu.sync_copy(pos_hbm, pos_v)
    pltpu.sync_copy(srcv_hbm, srcv_v)
    pltpu.sync_copy(w_hbm, w_v)
    zi = jnp.zeros((L,), jnp.int32)
    zf = jnp.zeros((L,), jnp.float32)
    for j in range(SL // L):
        idx_v[pl.ds(j * L, L)] = zi
        sw_v[pl.ds(j * L, L)] = zf

    def scan(i, _):
        p = pos_v[pl.ds(i * L, L)]
        sv = srcv_v[pl.ds(i * L, L)]
        wv = w_v[pl.ds(i * L, L)]
        inrange = (p >= base) & (p < base + SL)
        loc = jnp.where(inrange, p - base, 0)
        plsc.store_scatter(idx_v, [loc], sv, mask=inrange)
        plsc.store_scatter(sw_v, [loc], wv, mask=inrange)
        return 0

    lax.fori_loop(0, (B * K) // L, scan, 0)
    pltpu.async_copy(xf_hbm.at[idx_v], rows_v, sem).wait()
    pltpu.sync_copy(rows_v, xg_hbm.at[pl.ds(base, SL)])
    pltpu.sync_copy(sw_v, sw_hbm.at[pl.ds(base, SL)])


def _ffn_body(eob_ref, used_ref, xg_ref, wi_ref, bi_ref, wo_ref, bo_ref,
              sw_ref, y_ref, h_ref):
    g = pl.program_id(0)

    @pl.when(g < used_ref[0])
    def _():
        x = xg_ref[...].astype(jnp.float32)
        h = lax.dot_general(x, wi_ref[0], (((1,), (1,)), ((), ())),
                            preferred_element_type=jnp.float32)
        h_ref[...] = jnp.maximum(h + bi_ref[0], 0.0)
        o = lax.dot_general(h_ref[...], wo_ref[0], (((1,), (1,)), ((), ())),
                            preferred_element_type=jnp.float32)
        y_ref[...] = (o + bo_ref[0]) * sw_ref[...]


def _combine_body(y_hbm, pos_hbm, out_hbm, idx_v, rows_v, out_v, sem):
    wid = lax.axis_index("s") * NC + lax.axis_index("c")
    for c in range(TB // CCH):
        tok0 = wid * TB + c * CCH
        pltpu.sync_copy(pos_hbm.at[pl.ds(tok0 * K, CCH * K)], idx_v)
        pltpu.async_copy(y_hbm.at[idx_v], rows_v, sem).wait()

        def addbody(i, _):
            t = i // (O // L)
            j = i - t * (O // L)
            a = rows_v[2 * t, pl.ds(j * L, L)]
            b = rows_v[2 * t + 1, pl.ds(j * L, L)]
            out_v[t, pl.ds(j * L, L)] = a + b
            return 0

        lax.fori_loop(0, CCH * (O // L), addbody, 0)
        pltpu.sync_copy(out_v, out_hbm.at[pl.ds(tok0, CCH)])


@functools.lru_cache(maxsize=1)
def _sc_mesh():
    return plsc.VectorSubcoreMesh(core_axis_name="c", subcore_axis_name="s",
                                  num_cores=NC, num_subcores=NS)


@jax.jit
def kernel(trait_0, trait_1, trait_2, trait_3, trait_4, trait_5, trait_6,
           trait_7, gate_W, gate_b, W_in, b_in, W_out, b_out):
    traits = [trait_0, trait_1, trait_2, trait_3, trait_4, trait_5,
              trait_6, trait_7]

    logits = pl.pallas_call(
        _gate_body,
        grid=(B // GB,),
        in_specs=[pl.BlockSpec((GB, D), lambda i: (i, 0))
                  for _ in range(E)] + [
            pl.BlockSpec((E, D * E), lambda i: (0, 0)),
            pl.BlockSpec((E,), lambda i: (0,)),
        ],
        out_specs=pl.BlockSpec((GB, E), lambda i: (i, 0)),
        out_shape=jax.ShapeDtypeStruct((B, E), jnp.float32),
    )(*traits, gate_W, gate_b)

    pos2, srcv2, wv2, eob_row, used11 = pl.pallas_call(
        _route_body,
        out_shape=[
            jax.ShapeDtypeStruct((B, K), jnp.int32),
            jax.ShapeDtypeStruct((B, K), jnp.int32),
            jax.ShapeDtypeStruct((B, K), jnp.float32),
            jax.ShapeDtypeStruct((1, NB), jnp.int32),
            jax.ShapeDtypeStruct((1, 1), jnp.int32),
        ],
    )(logits)

    pos_flat = pos2.reshape(B * K)
    srcv_flat = srcv2.reshape(B * K)
    w_flat = wv2.reshape(B * K)
    xb = jnp.stack(traits, axis=0).astype(jnp.bfloat16)
    xf = lax.bitcast_convert_type(
        xb.reshape(E * B, DW, 2), jnp.int32)

    xg_i, sw = pl.kernel(
        _route_gather_body,
        out_type=[
            jax.ShapeDtypeStruct((CT, DW), jnp.int32),
            jax.ShapeDtypeStruct((CT,), jnp.float32),
        ],
        mesh=_sc_mesh(),
        compiler_params=pltpu.CompilerParams(needs_layout_passes=False),
        scratch_types=[
            pltpu.VMEM((B * K,), jnp.int32),
            pltpu.VMEM((B * K,), jnp.int32),
            pltpu.VMEM((B * K,), jnp.float32),
            pltpu.VMEM((SL,), jnp.int32),
            pltpu.VMEM((SL,), jnp.float32),
            pltpu.VMEM((SL, DW), jnp.int32),
            pltpu.SemaphoreType.DMA,
        ],
    )(pos_flat, srcv_flat, w_flat, xf)

    grid_spec = pltpu.PrefetchScalarGridSpec(
        num_scalar_prefetch=2,
        grid=(NB,),
        in_specs=[
            pl.BlockSpec((BT, D), lambda g, eob, used: (g, 0)),
            pl.BlockSpec((1, H, D), lambda g, eob, used: (eob[g], 0, 0)),
            pl.BlockSpec((1, 1, H), lambda g, eob, used: (eob[g], 0, 0)),
            pl.BlockSpec((1, O, H), lambda g, eob, used: (eob[g], 0, 0)),
            pl.BlockSpec((1, 1, O), lambda g, eob, used: (eob[g], 0, 0)),
            pl.BlockSpec((BT, 1), lambda g, eob, used: (g, 0)),
        ],
        out_specs=pl.BlockSpec((BT, O), lambda g, eob, used: (g, 0)),
        scratch_shapes=[pltpu.VMEM((BT, H), jnp.float32)],
    )
    eob_clamped = jnp.minimum(eob_row.reshape(NB), E - 1)
    xg_bf = lax.bitcast_convert_type(xg_i, jnp.bfloat16).reshape(CT, D)
    y = pl.pallas_call(
        _ffn_body,
        grid_spec=grid_spec,
        out_shape=jax.ShapeDtypeStruct((CT, O), jnp.float32),
    )(eob_clamped, used11.reshape(1), xg_bf, W_in,
      b_in.reshape(E, 1, H), W_out,
      b_out.reshape(E, 1, O), sw.reshape(CT, 1))

    out = pl.kernel(
        _combine_body,
        out_type=jax.ShapeDtypeStruct((B, O), jnp.float32),
        mesh=_sc_mesh(),
        compiler_params=pltpu.CompilerParams(needs_layout_passes=False),
        scratch_types=[
            pltpu.VMEM((CCH * K,), jnp.int32),
            pltpu.VMEM((CCH * K, O), jnp.float32),
            pltpu.VMEM((CCH, O), jnp.float32),
            pltpu.SemaphoreType.DMA,
        ],
    )(y, pos_flat)
    return out


# speedup vs baseline: 1.0477x; 1.0477x over previous
import functools

import jax
import jax.numpy as jnp
from jax import lax
from jax.experimental import pallas as pl
from jax.experimental.pallas import tpu as pltpu
from jax.experimental.pallas import tpu_sc as plsc

B, D, H, O, E = 2048, 1024, 2048, 1024, 8
K = 2
BT = 256
NB = (B * K) // BT + E
CT = NB * BT
NC, NS, L = 2, 16, 16
NW = NC * NS
SL = CT // NW
TB = B // NW
CCH = TB // 2
GB = 128
DW = D // 2


def _gate_body(*refs):
    x_refs = refs[:E]
    gw_ref, gb_ref, lg_ref, xb_ref = refs[E], refs[E + 1], refs[E + 2], refs[E + 3]
    acc = gb_ref[...][None, :]
    for j in range(E):
        x = x_refs[j][...]
        gw = gw_ref[:, j * D:(j + 1) * D]
        acc = acc + lax.dot_general(x, gw,
                                    (((1,), (1,)), ((), ())),
                                    preferred_element_type=jnp.float32)
        xb_ref[j] = x.astype(jnp.bfloat16)
    lg_ref[...] = acc


def _route_body(lg_ref, pos_ref, srcv_ref, wv_ref, eob_ref, used_ref):
    l = lg_ref[...]
    cols = lax.broadcasted_iota(jnp.int32, (B, E), 1)
    a1 = jnp.argmax(l, axis=1)[:, None]
    m1 = jnp.max(l, axis=1)[:, None]
    lmask = jnp.where(cols == a1, -jnp.inf, l)
    a2 = jnp.argmax(lmask, axis=1)[:, None]
    m2 = jnp.max(lmask, axis=1)[:, None]
    z = jnp.exp(m2 - m1)
    w1 = 1.0 / (1.0 + z)
    w2 = 1.0 - w1
    msk1 = (cols == a1).astype(jnp.float32)
    msk2 = (cols == a2).astype(jnp.float32)
    m = msk1 + msk2
    rows_i = lax.broadcasted_iota(jnp.int32, (B, B), 0)
    cols_i = lax.broadcasted_iota(jnp.int32, (B, B), 1)
    lt = (rows_i > cols_i).astype(jnp.float32)
    rank = lax.dot_general(lt, m, (((1,), (0,)), ((), ())),
                           preferred_element_type=jnp.float32)
    counts = jnp.sum(m, axis=0)[None, :]
    nb = jnp.floor((counts + (BT - 1)) * (1.0 / BT))
    r8 = lax.broadcasted_iota(jnp.int32, (E, E), 0)
    c8 = lax.broadcasted_iota(jnp.int32, (E, E), 1)
    t8 = (r8 <= c8).astype(jnp.float32)
    cum = lax.dot_general(nb, t8, (((1,), (0,)), ((), ())),
                          preferred_element_type=jnp.float32)
    off = (cum - nb) * float(BT)
    pos_e = off + rank
    pos0 = jnp.sum(msk1 * pos_e, axis=1)[:, None]
    pos1 = jnp.sum(msk2 * pos_e, axis=1)[:, None]
    pos_ref[...] = jnp.concatenate(
        [pos0.astype(jnp.int32), pos1.astype(jnp.int32)], axis=1)
    t_iota = lax.broadcasted_iota(jnp.int32, (B, 1), 0)
    srcv_ref[...] = jnp.concatenate(
        [a1 * B + t_iota, a2 * B + t_iota], axis=1)
    wv_ref[...] = jnp.concatenate([w1, w2], axis=1)
    g_iota = lax.broadcasted_iota(jnp.int32, (1, NB), 1).astype(jnp.float32)
    eob = jnp.zeros((1, NB), jnp.float32)
    for j in range(E - 1):
        cj = lax.slice(cum, (0, j), (1, j + 1))
        eob = eob + (g_iota >= cj).astype(jnp.float32)
    eob_ref[...] = eob.astype(jnp.int32)
    used_ref[...] = lax.slice(cum, (0, E - 1), (1, E)).astype(jnp.int32)


def _route_gather_body(pos_hbm, srcv_hbm, w_hbm, xf_hbm, xg_hbm, sw_hbm,
                       pos_v, srcv_v, w_v, idx_v, sw_v, rows_v, sem):
    wid = lax.axis_index("s") * NC + lax.axis_index("c")
    base = wid * SL
    pltpu.sync_copy(pos_hbm, pos_v)
    pltpu.sync_copy(srcv_hbm, srcv_v)
    pltpu.sync_copy(w_hbm, w_v)
    zi = jnp.zeros((L,), jnp.int32)
    zf = jnp.zeros((L,), jnp.float32)
    for j in range(SL // L):
        idx_v[pl.ds(j * L, L)] = zi
        sw_v[pl.ds(j * L, L)] = zf

    def scan(i, _):
        p = pos_v[pl.ds(i * L, L)]
        sv = srcv_v[pl.ds(i * L, L)]
        wv = w_v[pl.ds(i * L, L)]
        inrange = (p >= base) & (p < base + SL)
        loc = jnp.where(inrange, p - base, 0)
        plsc.store_scatter(idx_v, [loc], sv, mask=inrange)
        plsc.store_scatter(sw_v, [loc], wv, mask=inrange)
        return 0

    lax.fori_loop(0, (B * K) // L, scan, 0)
    pltpu.async_copy(xf_hbm.at[idx_v], rows_v, sem).wait()
    pltpu.sync_copy(rows_v, xg_hbm.at[pl.ds(base, SL)])
    pltpu.sync_copy(sw_v, sw_hbm.at[pl.ds(base, SL)])


def _ffn_body(eob_ref, used_ref, xg_ref, wi_ref, bi_ref, wo_ref, bo_ref,
              sw_ref, y_ref, h_ref):
    g = pl.program_id(0)

    @pl.when(g < used_ref[0])
    def _():
        x = xg_ref[...]
        wi = wi_ref[0].astype(jnp.bfloat16)
        h = lax.dot_general(x, wi, (((1,), (1,)), ((), ())),
                            preferred_element_type=jnp.float32)
        h_ref[...] = jnp.maximum(h + bi_ref[0], 0.0).astype(jnp.bfloat16)
        wo = wo_ref[0].astype(jnp.bfloat16)
        o = lax.dot_general(h_ref[...], wo, (((1,), (1,)), ((), ())),
                            preferred_element_type=jnp.float32)
        y_ref[...] = (o + bo_ref[0]) * sw_ref[...]


def _combine_body(y_hbm, pos_hbm, out_hbm, idx_v, rows_v, out_v, sem):
    wid = lax.axis_index("s") * NC + lax.axis_index("c")
    for c in range(TB // CCH):
        tok0 = wid * TB + c * CCH
        pltpu.sync_copy(pos_hbm.at[pl.ds(tok0 * K, CCH * K)], idx_v)
        pltpu.async_copy(y_hbm.at[idx_v], rows_v, sem).wait()

        def addbody(i, _):
            t = i // (O // L)
            j = i - t * (O // L)
            a = rows_v[2 * t, pl.ds(j * L, L)]
            b = rows_v[2 * t + 1, pl.ds(j * L, L)]
            out_v[t, pl.ds(j * L, L)] = a + b
            return 0

        lax.fori_loop(0, CCH * (O // L), addbody, 0)
        pltpu.sync_copy(out_v, out_hbm.at[pl.ds(tok0, CCH)])


@functools.lru_cache(maxsize=1)
def _sc_mesh():
    return plsc.VectorSubcoreMesh(core_axis_name="c", subcore_axis_name="s",
                                  num_cores=NC, num_subcores=NS)


@jax.jit
def kernel(trait_0, trait_1, trait_2, trait_3, trait_4, trait_5, trait_6,
           trait_7, gate_W, gate_b, W_in, b_in, W_out, b_out):
    traits = [trait_0, trait_1, trait_2, trait_3, trait_4, trait_5,
              trait_6, trait_7]

    logits, xb = pl.pallas_call(
        _gate_body,
        grid=(B // GB,),
        in_specs=[pl.BlockSpec((GB, D), lambda i: (i, 0))
                  for _ in range(E)] + [
            pl.BlockSpec((E, D * E), lambda i: (0, 0)),
            pl.BlockSpec((E,), lambda i: (0,)),
        ],
        out_specs=[
            pl.BlockSpec((GB, E), lambda i: (i, 0)),
            pl.BlockSpec((E, GB, D), lambda i: (0, i, 0)),
        ],
        out_shape=[
            jax.ShapeDtypeStruct((B, E), jnp.float32),
            jax.ShapeDtypeStruct((E, B, D), jnp.bfloat16),
        ],
    )(*traits, gate_W, gate_b)

    pos2, srcv2, wv2, eob_row, used11 = pl.pallas_call(
        _route_body,
        out_shape=[
            jax.ShapeDtypeStruct((B, K), jnp.int32),
            jax.ShapeDtypeStruct((B, K), jnp.int32),
            jax.ShapeDtypeStruct((B, K), jnp.float32),
            jax.ShapeDtypeStruct((1, NB), jnp.int32),
            jax.ShapeDtypeStruct((1, 1), jnp.int32),
        ],
    )(logits)

    pos_flat = pos2.reshape(B * K)
    srcv_flat = srcv2.reshape(B * K)
    w_flat = wv2.reshape(B * K)
    xf = lax.bitcast_convert_type(
        xb.reshape(E * B, DW, 2), jnp.int32)

    xg_i, sw = pl.kernel(
        _route_gather_body,
        out_type=[
            jax.ShapeDtypeStruct((CT, DW), jnp.int32),
            jax.ShapeDtypeStruct((CT,), jnp.float32),
        ],
        mesh=_sc_mesh(),
        compiler_params=pltpu.CompilerParams(needs_layout_passes=False),
        scratch_types=[
            pltpu.VMEM((B * K,), jnp.int32),
            pltpu.VMEM((B * K,), jnp.int32),
            pltpu.VMEM((B * K,), jnp.float32),
            pltpu.VMEM((SL,), jnp.int32),
            pltpu.VMEM((SL,), jnp.float32),
            pltpu.VMEM((SL, DW), jnp.int32),
            pltpu.SemaphoreType.DMA,
        ],
    )(pos_flat, srcv_flat, w_flat, xf)

    grid_spec = pltpu.PrefetchScalarGridSpec(
        num_scalar_prefetch=2,
        grid=(NB,),
        in_specs=[
            pl.BlockSpec((BT, D), lambda g, eob, used: (g, 0)),
            pl.BlockSpec((1, H, D), lambda g, eob, used: (eob[g], 0, 0)),
            pl.BlockSpec((1, 1, H), lambda g, eob, used: (eob[g], 0, 0)),
            pl.BlockSpec((1, O, H), lambda g, eob, used: (eob[g], 0, 0)),
            pl.BlockSpec((1, 1, O), lambda g, eob, used: (eob[g], 0, 0)),
            pl.BlockSpec((BT, 1), lambda g, eob, used: (g, 0)),
        ],
        out_specs=pl.BlockSpec((BT, O), lambda g, eob, used: (g, 0)),
        scratch_shapes=[pltpu.VMEM((BT, H), jnp.bfloat16)],
    )
    eob_clamped = jnp.minimum(eob_row.reshape(NB), E - 1)
    xg = lax.bitcast_convert_type(xg_i, jnp.bfloat16).reshape(CT, D)
    y = pl.pallas_call(
        _ffn_body,
        grid_spec=grid_spec,
        out_shape=jax.ShapeDtypeStruct((CT, O), jnp.float32),
    )(eob_clamped, used11.reshape(1), xg, W_in,
      b_in.reshape(E, 1, H), W_out,
      b_out.reshape(E, 1, O), sw.reshape(CT, 1))

    out = pl.kernel(
        _combine_body,
        out_type=jax.ShapeDtypeStruct((B, O), jnp.float32),
        mesh=_sc_mesh(),
        compiler_params=pltpu.CompilerParams(needs_layout_passes=False),
        scratch_types=[
            pltpu.VMEM((CCH * K,), jnp.int32),
            pltpu.VMEM((CCH * K, O), jnp.float32),
            pltpu.VMEM((CCH, O), jnp.float32),
            pltpu.SemaphoreType.DMA,
        ],
    )(y, pos_flat)
    return out


# speedup vs baseline: 4.3950x; 4.1950x over previous
import functools

import jax
import jax.numpy as jnp
from jax import lax
from jax.experimental import pallas as pl
from jax.experimental.pallas import tpu as pltpu

B, D, H, O, E = 2048, 1024, 2048, 1024, 8
K = 2
BT = 256
NB = (B * K) // BT + E
GB = 128


def _gate_body(*refs):
    x_refs = refs[:E]
    gw_ref, gb_ref, lg_ref, xb_ref = (refs[E], refs[E + 1], refs[E + 2],
                                      refs[E + 3])
    acc = gb_ref[...][None, :]
    for j in range(E):
        x = x_refs[j][...]
        gw = gw_ref[:, j * D:(j + 1) * D]
        acc = acc + lax.dot_general(x, gw, (((1,), (1,)), ((), ())),
                                    preferred_element_type=jnp.float32)
        xb_ref[j] = x.astype(jnp.bfloat16)
    lg_ref[...] = acc


def _route_body(lg_ref, pt_ref, eob_ref, used_ref):
    l = lg_ref[...]
    cols = lax.broadcasted_iota(jnp.int32, (B, E), 1)
    a1 = jnp.argmax(l, axis=1)[:, None]
    m1 = jnp.max(l, axis=1)[:, None]
    lmask = jnp.where(cols == a1, -jnp.inf, l)
    a2 = jnp.argmax(lmask, axis=1)[:, None]
    m2 = jnp.max(lmask, axis=1)[:, None]
    z = jnp.exp(m2 - m1)
    w1 = 1.0 / (1.0 + z)
    w2 = 1.0 - w1
    msk1 = (cols == a1).astype(jnp.float32)
    msk2 = (cols == a2).astype(jnp.float32)
    m = msk1 + msk2
    rows_i = lax.broadcasted_iota(jnp.int32, (B, B), 0)
    cols_i = lax.broadcasted_iota(jnp.int32, (B, B), 1)
    lt = (rows_i > cols_i).astype(jnp.float32)
    rank = lax.dot_general(lt, m, (((1,), (0,)), ((), ())),
                           precision=lax.Precision.HIGHEST,
                           preferred_element_type=jnp.float32)
    counts = jnp.sum(m, axis=0)[None, :]
    nb = jnp.floor((counts + (BT - 1)) * (1.0 / BT))
    r8 = lax.broadcasted_iota(jnp.int32, (E, E), 0)
    c8 = lax.broadcasted_iota(jnp.int32, (E, E), 1)
    t8 = (r8 <= c8).astype(jnp.float32)
    cum = lax.dot_general(nb, t8, (((1,), (0,)), ((), ())),
                          precision=lax.Precision.HIGHEST,
                          preferred_element_type=jnp.float32)
    off = (cum - nb) * float(BT)
    pos_e = off + rank
    pos0 = jnp.sum(msk1 * pos_e, axis=1)[:, None]
    pos1 = jnp.sum(msk2 * pos_e, axis=1)[:, None]
    m4 = jnp.concatenate([pos0, pos1, w1, w2], axis=1)
    ident = (rows_i == cols_i).astype(jnp.float32)
    pt_ref[...] = lax.dot_general(m4, ident, (((0,), (0,)), ((), ())),
                                  precision=lax.Precision.HIGHEST,
                                  preferred_element_type=jnp.float32)
    g_iota = lax.broadcasted_iota(jnp.int32, (1, NB), 1).astype(jnp.float32)
    eob = jnp.zeros((1, NB), jnp.float32)
    for j in range(E - 1):
        cj = lax.slice(cum, (0, j), (1, j + 1))
        eob = eob + (g_iota >= cj).astype(jnp.float32)
    eob_ref[...] = eob.astype(jnp.int32)
    used_ref[...] = lax.slice(cum, (0, E - 1), (1, E)).astype(jnp.int32)


def _moe_body(eob_ref, used_ref, xb_ref, wi_ref, bi_ref, wo_ref, bo_ref,
              pt_ref, out_ref, h_ref):
    g = pl.program_id(0)

    @pl.when(g == 0)
    def _():
        out_ref[...] = jnp.zeros((B, O), jnp.float32)

    @pl.when(g < used_ref[0])
    def _():
        srow = (lax.broadcasted_iota(jnp.int32, (BT, B), 0)
                + g * BT).astype(jnp.float32)
        p0 = pt_ref[0:1, :]
        p1 = pt_ref[1:2, :]
        w0 = pt_ref[2:3, :]
        w1 = pt_ref[3:4, :]
        m0 = (srow == p0).astype(jnp.float32)
        m1 = (srow == p1).astype(jnp.float32)
        pg = (m0 + m1).astype(jnp.bfloat16)
        xd = lax.dot_general(pg, xb_ref[0], (((1,), (0,)), ((), ())),
                             preferred_element_type=jnp.float32)
        xd = xd.astype(jnp.bfloat16)
        wi = wi_ref[0].astype(jnp.bfloat16)
        h = lax.dot_general(xd, wi, (((1,), (1,)), ((), ())),
                            preferred_element_type=jnp.float32)
        h_ref[...] = jnp.maximum(h + bi_ref[0], 0.0).astype(jnp.bfloat16)
        wo = wo_ref[0].astype(jnp.bfloat16)
        y = lax.dot_general(h_ref[...], wo, (((1,), (1,)), ((), ())),
                            preferred_element_type=jnp.float32)
        y = (y + bo_ref[0]).astype(jnp.bfloat16)
        pw = (m0 * w0 + m1 * w1).astype(jnp.bfloat16)
        out_ref[...] += lax.dot_general(pw, y, (((0,), (0,)), ((), ())),
                                        preferred_element_type=jnp.float32)


@jax.jit
def kernel(trait_0, trait_1, trait_2, trait_3, trait_4, trait_5, trait_6,
           trait_7, gate_W, gate_b, W_in, b_in, W_out, b_out):
    traits = [trait_0, trait_1, trait_2, trait_3, trait_4, trait_5,
              trait_6, trait_7]

    logits, xb = pl.pallas_call(
        _gate_body,
        grid=(B // GB,),
        in_specs=[pl.BlockSpec((GB, D), lambda i: (i, 0))
                  for _ in range(E)] + [
            pl.BlockSpec((E, D * E), lambda i: (0, 0)),
            pl.BlockSpec((E,), lambda i: (0,)),
        ],
        out_specs=[
            pl.BlockSpec((GB, E), lambda i: (i, 0)),
            pl.BlockSpec((E, GB, D), lambda i: (0, i, 0)),
        ],
        out_shape=[
            jax.ShapeDtypeStruct((B, E), jnp.float32),
            jax.ShapeDtypeStruct((E, B, D), jnp.bfloat16),
        ],
    )(*traits, gate_W, gate_b)

    pt, eob_row, used11 = pl.pallas_call(
        _route_body,
        out_shape=[
            jax.ShapeDtypeStruct((4, B), jnp.float32),
            jax.ShapeDtypeStruct((1, NB), jnp.int32),
            jax.ShapeDtypeStruct((1, 1), jnp.int32),
        ],
    )(logits)

    grid_spec = pltpu.PrefetchScalarGridSpec(
        num_scalar_prefetch=2,
        grid=(NB,),
        in_specs=[
            pl.BlockSpec((1, B, D), lambda g, eob, used: (eob[g], 0, 0)),
            pl.BlockSpec((1, H, D), lambda g, eob, used: (eob[g], 0, 0)),
            pl.BlockSpec((1, 1, H), lambda g, eob, used: (eob[g], 0, 0)),
            pl.BlockSpec((1, O, H), lambda g, eob, used: (eob[g], 0, 0)),
            pl.BlockSpec((1, 1, O), lambda g, eob, used: (eob[g], 0, 0)),
            pl.BlockSpec((4, B), lambda g, eob, used: (0, 0)),
        ],
        out_specs=pl.BlockSpec((B, O), lambda g, eob, used: (0, 0)),
        scratch_shapes=[pltpu.VMEM((BT, H), jnp.bfloat16)],
    )
    eob_clamped = jnp.minimum(eob_row.reshape(NB), E - 1)
    out = pl.pallas_call(
        _moe_body,
        grid_spec=grid_spec,
        out_shape=jax.ShapeDtypeStruct((B, O), jnp.float32),
    )(eob_clamped, used11.reshape(1), xb, W_in, b_in.reshape(E, 1, H),
      W_out, b_out.reshape(E, 1, O), pt)
    return out
